# baseline (speedup 1.0000x reference)
import functools

import jax
import jax.numpy as jnp
from jax import lax
from jax.experimental import pallas as pl
from jax.experimental.pallas import tpu as pltpu
from jax.experimental.pallas import tpu_sc as plsc

N = 10000
E = 320000
D = 128
D_EDGE = 12
L = 5
G = 64

NC = 2
NS = 16
NW = NC * NS
EPW = E // NW

CH1 = 2000
CH2 = 2000
RB = 80
NSB = CH2 // RB

NB = 2000
EB = 3200


def _pre_body(ea_ref, we_ref, aev_ref, out_ref):
    cols = [jnp.dot(we_ref[l], aev_ref[l][:, None],
                    preferred_element_type=jnp.float32) for l in range(L)]
    cols.append(jnp.zeros((D_EDGE, 8 - L), jnp.float32))
    w = jnp.concatenate(cols, axis=1)
    out_ref[...] = lax.dot_general(w, ea_ref[...], (((0,), (1,)), ((), ())),
                                   preferred_element_type=jnp.float32)


def _edge_alpha_pre(edge_attr, W_e, a_e):
    return pl.pallas_call(
        _pre_body,
        grid=(E // EB,),
        in_specs=[
            pl.BlockSpec((EB, D_EDGE), lambda i: (i, 0)),
            pl.BlockSpec((L, D_EDGE, D), lambda i: (0, 0, 0)),
            pl.BlockSpec((L, D), lambda i: (0, 0)),
        ],
        out_specs=pl.BlockSpec((8, EB), lambda i: (0, i)),
        out_shape=jax.ShapeDtypeStruct((8, E), jnp.float32),
    )(edge_attr, W_e, a_e)


def _score0_body(x_ref, w_ref, a_ref, h_ref, s_ref):
    hb = jnp.dot(x_ref[...], w_ref[...], preferred_element_type=jnp.float32)
    h_ref[...] = hb
    s_ref[...] = jnp.dot(hb, a_ref[...], preferred_element_type=jnp.float32)


def _score_body(p0_ref, p1_ref, b_ref, w_ref, a_ref, h_ref, s_ref):
    xb = jnp.maximum(p0_ref[...] + p1_ref[...] + b_ref[...], 0.0)
    hb = jnp.dot(xb, w_ref[...], preferred_element_type=jnp.float32)
    h_ref[...] = hb
    s_ref[...] = jnp.dot(hb, a_ref[...], preferred_element_type=jnp.float32)


_OUT_HS = (jax.ShapeDtypeStruct((N, D), jnp.float32),
           jax.ShapeDtypeStruct((N, 8), jnp.float32))
_HS_SPECS = (pl.BlockSpec((NB, D), lambda i: (i, 0)),
             pl.BlockSpec((NB, 8), lambda i: (i, 0)))


def _score0(x, Wl, A):
    return pl.pallas_call(
        _score0_body,
        grid=(N // NB,),
        in_specs=[
            pl.BlockSpec((NB, D), lambda i: (i, 0)),
            pl.BlockSpec((D, D), lambda i: (0, 0)),
            pl.BlockSpec((D, 8), lambda i: (0, 0)),
        ],
        out_specs=_HS_SPECS,
        out_shape=_OUT_HS,
    )(x, Wl, A)


def _score(p0, p1, bvec, Wl, A):
    return pl.pallas_call(
        _score_body,
        grid=(N // NB,),
        in_specs=[
            pl.BlockSpec((NB, D), lambda i: (i, 0)),
            pl.BlockSpec((NB, D), lambda i: (i, 0)),
            pl.BlockSpec((1, D), lambda i: (0, 0)),
            pl.BlockSpec((D, D), lambda i: (0, 0)),
            pl.BlockSpec((D, 8), lambda i: (0, 0)),
        ],
        out_specs=_HS_SPECS,
        out_shape=_OUT_HS,
    )(p0, p1, bvec, Wl, A)


def _fin_body(p0_ref, p1_ref, b_ref, bt_ref, wl_ref, bl_ref, out_ref,
              acc_ref, cnt_ref):
    i = pl.program_id(0)

    @pl.when(i == 0)
    def _():
        acc_ref[...] = jnp.zeros_like(acc_ref)
        cnt_ref[...] = jnp.zeros_like(cnt_ref)

    hb = p0_ref[...] + p1_ref[...] + b_ref[...]
    bt = bt_ref[0, 0, :]
    oh = (lax.broadcasted_iota(jnp.int32, (G, NB), 0)
          == bt[None, :]).astype(jnp.float32)
    acc_ref[...] += lax.dot_general(oh, hb, (((1,), (0,)), ((), ())),
                                    preferred_element_type=jnp.float32)
    cnt_ref[...] += jnp.broadcast_to(
        jnp.sum(oh, axis=1, keepdims=True), (G, D))

    @pl.when(i == pl.num_programs(0) - 1)
    def _():
        pooled = acc_ref[...] / jnp.maximum(cnt_ref[...], 1.0)
        out_ref[...] = (jnp.dot(pooled, wl_ref[...],
                                preferred_element_type=jnp.float32)
                        + bl_ref[...])


def _final(p0, p1, bvec, bt3, Wlin_pad, blin_pad):
    return pl.pallas_call(
        _fin_body,
        grid=(N // NB,),
        in_specs=[
            pl.BlockSpec((NB, D), lambda i: (i, 0)),
            pl.BlockSpec((NB, D), lambda i: (i, 0)),
            pl.BlockSpec((1, D), lambda i: (0, 0)),
            pl.BlockSpec((1, 1, NB), lambda i: (i, 0, 0)),
            pl.BlockSpec((D, 8), lambda i: (0, 0)),
            pl.BlockSpec((1, 8), lambda i: (0, 0)),
        ],
        out_specs=pl.BlockSpec((G, 8), lambda i: (0, 0)),
        out_shape=jax.ShapeDtypeStruct((G, 8), jnp.float32),
        scratch_shapes=[pltpu.VMEM((G, D), jnp.float32),
                        pltpu.VMEM((G, D), jnp.float32)],
    )(p0, p1, bvec, bt3, Wlin_pad, blin_pad)


_MESH = plsc.VectorSubcoreMesh(core_axis_name="c", subcore_axis_name="s")
_SC_PARAMS = pltpu.CompilerParams(needs_layout_passes=False)


DW = 128
DR = 10240 // DW


@functools.partial(
    pl.kernel,
    out_type=[jax.ShapeDtypeStruct((E,), jnp.float32),
              jax.ShapeDtypeStruct((NC, DR, DW), jnp.float32)],
    mesh=_MESH,
    compiler_params=_SC_PARAMS,
    scratch_types=[
        pltpu.VMEM((N * 8,), jnp.float32),
        pltpu.VMEM((DR, DW), jnp.float32),
        pltpu.VMEM((DR,), jnp.int32),
        pltpu.VMEM((CH1,), jnp.int32),
        pltpu.VMEM((CH1,), jnp.int32),
        pltpu.VMEM((CH1,), jnp.float32),
        pltpu.VMEM((CH1,), jnp.float32),
        pltpu.VMEM_SHARED((DR, DW), jnp.float32),
    ],
)
def _edge_pass1(src_hbm, dst_hbm, ae_hbm, s_hbm, ex_hbm, den_hbm,
                s_vm, den_vm, idx_vm, src_vm, dst_vm, ae_vm, ex_vm,
                den_sh):
    cid = lax.axis_index("c")
    sid = lax.axis_index("s")
    wid = sid * NC + cid
    base = wid * EPW
    pltpu.sync_copy(s_hbm, s_vm)

    i16 = lax.iota(jnp.int32, 16)
    for g in range(DR // 16):
        idx_vm[pl.ds(g * 16, 16)] = i16 + g * 16

    def zero_body(i, carry):
        r = i // (DW // 16)
        k = i % (DW // 16)
        den_vm[r, pl.ds(k * 16, 16)] = jnp.zeros((16,), jnp.float32)
        return carry
    lax.fori_loop(0, DR * (DW // 16), zero_body, 0)

    @pl.when(sid < DR // 8)
    def _():
        pltpu.sync_copy(den_vm.at[pl.ds(0, 8), :],
                        den_sh.at[pl.ds(sid * 8, 8), :])
    plsc.subcore_barrier()

    def chunk_body(c, carry):
        off = base + c * CH1
        pltpu.sync_copy(src_hbm.at[pl.ds(off, CH1)], src_vm)
        pltpu.sync_copy(dst_hbm.at[pl.ds(off, CH1)], dst_vm)
        pltpu.sync_copy(ae_hbm.at[pl.ds(off, CH1)], ae_vm)

        @plsc.parallel_loop(0, CH1 // 16, unroll=4)
        def _grp(i):
            sl = pl.ds(i * 16, 16)
            si = src_vm[sl]
            di = dst_vm[sl]
            a = (plsc.load_gather(s_vm, [si * 8])
                 + plsc.load_gather(s_vm, [di * 8 + 1])
                 + ae_vm[sl])
            a = jnp.where(a >= 0.0, a, 0.2 * a)
            e = jnp.exp(a)
            ex_vm[sl] = e
            plsc.addupdate_scatter(den_vm, [di >> 7, di & 127], e)
        pltpu.sync_copy(ex_vm, ex_hbm.at[pl.ds(off, CH1)])
        return carry
    lax.fori_loop(0, EPW // CH1, chunk_body, 0)

    pltpu.sync_copy(den_vm, den_sh.at[idx_vm], add=True)
    plsc.subcore_barrier()

    @pl.when(sid < DR // 8)
    def _():
        pltpu.sync_copy(den_sh.at[pl.ds(sid * 8, 8), :],
                        den_hbm.at[cid, pl.ds(sid * 8, 8), :])


ZR = 40
NZG = N // ZR


@functools.partial(
    pl.kernel,
    out_type=jax.ShapeDtypeStruct((NC, N, D), jnp.float32),
    mesh=_MESH,
    compiler_params=_SC_PARAMS,
    scratch_types=[
        pltpu.VMEM((DR, DW), jnp.float32),
        pltpu.VMEM((CH2,), jnp.int32),
        pltpu.VMEM((CH2,), jnp.int32),
        pltpu.VMEM((CH2,), jnp.float32),
        pltpu.VMEM((CH2,), jnp.float32),
        pltpu.VMEM((RB, D), jnp.float32),
        pltpu.VMEM((RB, D), jnp.float32),
        pltpu.VMEM((RB,), jnp.int32),
        pltpu.VMEM((RB,), jnp.int32),
        pltpu.VMEM_SHARED((N, D), jnp.float32),
        pltpu.VMEM_SHARED((DR, DW), jnp.float32),
        pltpu.SemaphoreType.DMA,
        pltpu.SemaphoreType.DMA,
        pltpu.SemaphoreType.DMA,
        pltpu.SemaphoreType.DMA,
    ],
)
def _edge_pass2(src_hbm, dst_hbm, ex_hbm, den_hbm, h_hbm,
                parts_hbm, den_vm, src_vm, dst_vm, ex_vm,
                coef_vm, rows0_vm, rows1_vm,
                dsub0_vm, dsub1_vm, acc_sh, den_sh, gsem0, gsem1,
                ssem0, ssem1):
    cid = lax.axis_index("c")
    sid = lax.axis_index("s")
    wid = sid * NC + cid
    base = wid * EPW

    @pl.when(sid < DR // 8)
    def _():
        pltpu.sync_copy(den_hbm.at[0, pl.ds(sid * 8, 8), :],
                        den_vm.at[pl.ds(0, 8), :])
        pltpu.sync_copy(den_hbm.at[1, pl.ds(sid * 8, 8), :],
                        den_vm.at[pl.ds(8, 8), :])

        def dadd(i, carry):
            r = i // (DW // 16)
            k = i % (DW // 16)
            sl = pl.ds(k * 16, 16)
            den_vm[r, sl] = den_vm[r, sl] + den_vm[r + 8, sl]
            return carry
        lax.fori_loop(0, 8 * (DW // 16), dadd, 0)
        pltpu.sync_copy(den_vm.at[pl.ds(0, 8), :],
                        den_sh.at[pl.ds(sid * 8, 8), :])

    def zrow(i, carry):
        r = i // (D // 16)
        d = i % (D // 16)
        rows0_vm[r, pl.ds(d * 16, 16)] = jnp.zeros((16,), jnp.float32)
        return carry
    lax.fori_loop(0, ZR * (D // 16), zrow, 0)
    assert ZR <= RB

    def zg(k, carry):
        g = sid + k * NS

        @pl.when(g < NZG)
        def _():
            pltpu.sync_copy(rows0_vm.at[pl.ds(0, ZR), :],
                            acc_sh.at[pl.ds(g * ZR, ZR), :])
        return carry
    lax.fori_loop(0, (NZG + NS - 1) // NS, zg, 0)
    plsc.subcore_barrier()
    pltpu.sync_copy(den_sh, den_vm)

    gbufs = (rows0_vm, rows1_vm)
    dsubs = (dsub0_vm, dsub1_vm)
    gsems = (gsem0, gsem1)
    ssems = (ssem0, ssem1)

    def _scale_and_scatter(sb):
        buf = gbufs[sb % 2]
        dsub = dsubs[sb % 2]
        b0 = sb * RB

        @plsc.parallel_loop(0, RB, unroll=8)
        def _scl(e):
            cvec = plsc.load_gather(
                coef_vm, [jnp.full((16,), b0 + e, jnp.int32)])
            for d in range(D // 16):
                sl = pl.ds(d * 16, 16)
                buf[e, sl] = buf[e, sl] * cvec
        for k in range(RB // 16):
            sl = pl.ds(k * 16, 16)
            dsub[sl] = dst_vm[pl.ds(b0 + k * 16, 16)]
        return pltpu.async_copy(buf, acc_sh.at[dsub], ssems[sb % 2],
                                add=True)

    def chunk_body(c, carry):
        off = base + c * CH2
        pltpu.sync_copy(src_hbm.at[pl.ds(off, CH2)], src_vm)
        pltpu.sync_copy(dst_hbm.at[pl.ds(off, CH2)], dst_vm)
        pltpu.sync_copy(ex_hbm.at[pl.ds(off, CH2)], ex_vm)

        @plsc.parallel_loop(0, CH2 // 16, unroll=4)
        def _cgrp(i):
            sl = pl.ds(i * 16, 16)
            di = dst_vm[sl]
            dn = plsc.load_gather(den_vm, [di >> 7, di & 127])
            coef_vm[sl] = ex_vm[sl] / (dn + 1e-16)

        gd = [None] * NSB
        sd = [None] * NSB
        gd[0] = pltpu.async_copy(
            h_hbm.at[src_vm.at[pl.ds(0, RB)]], gbufs[0], gsems[0])
        for sb in range(NSB):
            if sb + 1 < NSB:
                if sb >= 1:
                    sd[sb - 1].wait()
                gd[sb + 1] = pltpu.async_copy(
                    h_hbm.at[src_vm.at[pl.ds((sb + 1) * RB, RB)]],
                    gbufs[(sb + 1) % 2], gsems[(sb + 1) % 2])
            gd[sb].wait()
            sd[sb] = _scale_and_scatter(sb)
        sd[NSB - 2].wait()
        sd[NSB - 1].wait()
        return carry
    lax.fori_loop(0, EPW // CH2, chunk_body, 0)

    plsc.subcore_barrier()

    def og(k, carry):
        g = sid + k * NS

        @pl.when(g < NZG)
        def _():
            pltpu.sync_copy(acc_sh.at[pl.ds(g * ZR, ZR), :],
                            parts_hbm.at[cid, pl.ds(g * ZR, ZR), :])
        return carry
    lax.fori_loop(0, (NZG + NS - 1) // NS, og, 0)


def kernel(x, edge_index, edge_attr, batch, W, a_src, a_dst, W_e, a_e, b,
           W_lin, b_lin):
    src = edge_index[0]
    dst = edge_index[1]

    AE = _edge_alpha_pre(edge_attr, W_e, a_e)

    Wlin_pad = jnp.zeros((D, 8), jnp.float32).at[:, 0].set(W_lin[:, 0])
    blin_pad = jnp.zeros((1, 8), jnp.float32).at[0, 0].set(b_lin[0])
    bt3 = batch.reshape(N // NB, 1, NB)

    parts = None
    for l in range(L):
        A = jnp.zeros((D, 8), jnp.float32)
        A = A.at[:, 0].set(a_src[l]).at[:, 1].set(a_dst[l])
        if l == 0:
            H, S = _score0(x, W[0], A)
        else:
            H, S = _score(parts[0], parts[1], b[l - 1][None, :], W[l], A)
        ex, denp = _edge_pass1(src, dst, AE[l], S.reshape(N * 8))
        parts = _edge_pass2(src, dst, ex, denp, H)

    out = _final(parts[0], parts[1], b[L - 1][None, :], bt3,
                 Wlin_pad, blin_pad)
    return out[:, 0:1]


# speedup vs baseline: 1.0163x; 1.0163x over previous
import functools

import jax
import jax.numpy as jnp
from jax import lax
from jax.experimental import pallas as pl
from jax.experimental.pallas import tpu as pltpu
from jax.experimental.pallas import tpu_sc as plsc

N = 10000
E = 320000
D = 128
D_EDGE = 12
L = 5
G = 64

NC = 2
NS = 16
NW = NC * NS
EPW = E // NW

CH1 = 2000
CH2 = 2000
RB = 80
NSB = CH2 // RB

NB = 2000
EB = 3200


def _pre_body(ea_ref, we_ref, aev_ref, out_ref):
    cols = [jnp.dot(we_ref[l], aev_ref[l][:, None],
                    preferred_element_type=jnp.float32) for l in range(L)]
    cols.append(jnp.zeros((D_EDGE, 8 - L), jnp.float32))
    w = jnp.concatenate(cols, axis=1)
    out_ref[...] = lax.dot_general(w, ea_ref[...], (((0,), (1,)), ((), ())),
                                   preferred_element_type=jnp.float32)


def _edge_alpha_pre(edge_attr, W_e, a_e):
    return pl.pallas_call(
        _pre_body,
        grid=(E // EB,),
        in_specs=[
            pl.BlockSpec((EB, D_EDGE), lambda i: (i, 0)),
            pl.BlockSpec((L, D_EDGE, D), lambda i: (0, 0, 0)),
            pl.BlockSpec((L, D), lambda i: (0, 0)),
        ],
        out_specs=pl.BlockSpec((8, EB), lambda i: (0, i)),
        out_shape=jax.ShapeDtypeStruct((8, E), jnp.float32),
    )(edge_attr, W_e, a_e)


def _score0_body(x_ref, w_ref, a_ref, h_ref, s_ref):
    hb = jnp.dot(x_ref[...], w_ref[...], preferred_element_type=jnp.float32)
    h_ref[...] = hb
    s_ref[...] = jnp.dot(hb, a_ref[...], preferred_element_type=jnp.float32)


def _score_body(p0_ref, p1_ref, b_ref, w_ref, a_ref, h_ref, s_ref):
    xb = jnp.maximum(p0_ref[...] + p1_ref[...] + b_ref[...], 0.0)
    hb = jnp.dot(xb, w_ref[...], preferred_element_type=jnp.float32)
    h_ref[...] = hb
    s_ref[...] = jnp.dot(hb, a_ref[...], preferred_element_type=jnp.float32)


_OUT_HS = (jax.ShapeDtypeStruct((N, D), jnp.float32),
           jax.ShapeDtypeStruct((N, 8), jnp.float32))
_HS_SPECS = (pl.BlockSpec((NB, D), lambda i: (i, 0)),
             pl.BlockSpec((NB, 8), lambda i: (i, 0)))


def _score0(x, Wl, A):
    return pl.pallas_call(
        _score0_body,
        grid=(N // NB,),
        in_specs=[
            pl.BlockSpec((NB, D), lambda i: (i, 0)),
            pl.BlockSpec((D, D), lambda i: (0, 0)),
            pl.BlockSpec((D, 8), lambda i: (0, 0)),
        ],
        out_specs=_HS_SPECS,
        out_shape=_OUT_HS,
    )(x, Wl, A)


def _score(p0, p1, bvec, Wl, A):
    return pl.pallas_call(
        _score_body,
        grid=(N // NB,),
        in_specs=[
            pl.BlockSpec((NB, D), lambda i: (i, 0)),
            pl.BlockSpec((NB, D), lambda i: (i, 0)),
            pl.BlockSpec((1, D), lambda i: (0, 0)),
            pl.BlockSpec((D, D), lambda i: (0, 0)),
            pl.BlockSpec((D, 8), lambda i: (0, 0)),
        ],
        out_specs=_HS_SPECS,
        out_shape=_OUT_HS,
    )(p0, p1, bvec, Wl, A)


def _fin_body(p0_ref, p1_ref, b_ref, bt_ref, wl_ref, bl_ref, out_ref,
              acc_ref, cnt_ref):
    i = pl.program_id(0)

    @pl.when(i == 0)
    def _():
        acc_ref[...] = jnp.zeros_like(acc_ref)
        cnt_ref[...] = jnp.zeros_like(cnt_ref)

    hb = p0_ref[...] + p1_ref[...] + b_ref[...]
    bt = bt_ref[0, 0, :]
    oh = (lax.broadcasted_iota(jnp.int32, (G, NB), 0)
          == bt[None, :]).astype(jnp.float32)
    acc_ref[...] += lax.dot_general(oh, hb, (((1,), (0,)), ((), ())),
                                    preferred_element_type=jnp.float32)
    cnt_ref[...] += jnp.broadcast_to(
        jnp.sum(oh, axis=1, keepdims=True), (G, D))

    @pl.when(i == pl.num_programs(0) - 1)
    def _():
        pooled = acc_ref[...] / jnp.maximum(cnt_ref[...], 1.0)
        out_ref[...] = (jnp.dot(pooled, wl_ref[...],
                                preferred_element_type=jnp.float32)
                        + bl_ref[...])


def _final(p0, p1, bvec, bt3, Wlin_pad, blin_pad):
    return pl.pallas_call(
        _fin_body,
        grid=(N // NB,),
        in_specs=[
            pl.BlockSpec((NB, D), lambda i: (i, 0)),
            pl.BlockSpec((NB, D), lambda i: (i, 0)),
            pl.BlockSpec((1, D), lambda i: (0, 0)),
            pl.BlockSpec((1, 1, NB), lambda i: (i, 0, 0)),
            pl.BlockSpec((D, 8), lambda i: (0, 0)),
            pl.BlockSpec((1, 8), lambda i: (0, 0)),
        ],
        out_specs=pl.BlockSpec((G, 8), lambda i: (0, 0)),
        out_shape=jax.ShapeDtypeStruct((G, 8), jnp.float32),
        scratch_shapes=[pltpu.VMEM((G, D), jnp.float32),
                        pltpu.VMEM((G, D), jnp.float32)],
    )(p0, p1, bvec, bt3, Wlin_pad, blin_pad)


_MESH = plsc.VectorSubcoreMesh(core_axis_name="c", subcore_axis_name="s")
_SC_PARAMS = pltpu.CompilerParams(needs_layout_passes=False)


DW = 128
DR = 10240 // DW


@functools.partial(
    pl.kernel,
    out_type=[jax.ShapeDtypeStruct((E,), jnp.float32),
              jax.ShapeDtypeStruct((NC, DR, DW), jnp.float32)],
    mesh=_MESH,
    compiler_params=_SC_PARAMS,
    scratch_types=[
        pltpu.VMEM((N * 8,), jnp.float32),
        pltpu.VMEM((DR, DW), jnp.float32),
        pltpu.VMEM((DR,), jnp.int32),
        pltpu.VMEM((CH1,), jnp.int32),
        pltpu.VMEM((CH1,), jnp.int32),
        pltpu.VMEM((CH1,), jnp.float32),
        pltpu.VMEM((CH1,), jnp.float32),
        pltpu.VMEM_SHARED((DR, DW), jnp.float32),
    ],
)
def _edge_pass1(src_hbm, dst_hbm, ae_hbm, s_hbm, ex_hbm, den_hbm,
                s_vm, den_vm, idx_vm, src_vm, dst_vm, ae_vm, ex_vm,
                den_sh):
    cid = lax.axis_index("c")
    sid = lax.axis_index("s")
    wid = sid * NC + cid
    base = wid * EPW
    pltpu.sync_copy(s_hbm, s_vm)

    i16 = lax.iota(jnp.int32, 16)
    for g in range(DR // 16):
        idx_vm[pl.ds(g * 16, 16)] = i16 + g * 16

    def zero_body(i, carry):
        r = i // (DW // 16)
        k = i % (DW // 16)
        den_vm[r, pl.ds(k * 16, 16)] = jnp.zeros((16,), jnp.float32)
        return carry
    lax.fori_loop(0, DR * (DW // 16), zero_body, 0)

    @pl.when(sid < DR // 8)
    def _():
        pltpu.sync_copy(den_vm.at[pl.ds(0, 8), :],
                        den_sh.at[pl.ds(sid * 8, 8), :])
    plsc.subcore_barrier()

    def chunk_body(c, carry):
        off = base + c * CH1
        pltpu.sync_copy(src_hbm.at[pl.ds(off, CH1)], src_vm)
        pltpu.sync_copy(dst_hbm.at[pl.ds(off, CH1)], dst_vm)
        pltpu.sync_copy(ae_hbm.at[pl.ds(off, CH1)], ae_vm)

        @plsc.parallel_loop(0, CH1 // 16, unroll=4)
        def _grp(i):
            sl = pl.ds(i * 16, 16)
            si = src_vm[sl]
            di = dst_vm[sl]
            a = (plsc.load_gather(s_vm, [si * 8])
                 + plsc.load_gather(s_vm, [di * 8 + 1])
                 + ae_vm[sl])
            a = jnp.where(a >= 0.0, a, 0.2 * a)
            e = jnp.exp(a)
            ex_vm[sl] = e
            plsc.addupdate_scatter(den_vm, [di >> 7, di & 127], e)
        pltpu.sync_copy(ex_vm, ex_hbm.at[pl.ds(off, CH1)])
        return carry
    lax.fori_loop(0, EPW // CH1, chunk_body, 0)

    pltpu.sync_copy(den_vm, den_sh.at[idx_vm], add=True)
    plsc.subcore_barrier()

    @pl.when(sid < DR // 8)
    def _():
        pltpu.sync_copy(den_sh.at[pl.ds(sid * 8, 8), :],
                        den_hbm.at[cid, pl.ds(sid * 8, 8), :])


ZR = 40
NZG = N // ZR


@functools.partial(
    pl.kernel,
    out_type=jax.ShapeDtypeStruct((NC, N, D), jnp.float32),
    mesh=_MESH,
    compiler_params=_SC_PARAMS,
    scratch_types=[
        pltpu.VMEM((DR, DW), jnp.float32),
        pltpu.VMEM((CH2,), jnp.int32),
        pltpu.VMEM((CH2,), jnp.int32),
        pltpu.VMEM((CH2,), jnp.float32),
        pltpu.VMEM((CH2,), jnp.float32),
        pltpu.VMEM((RB, D), jnp.float32),
        pltpu.VMEM((RB, D), jnp.float32),
        pltpu.VMEM((RB,), jnp.int32),
        pltpu.VMEM((RB,), jnp.int32),
        pltpu.VMEM_SHARED((N, D), jnp.float32),
        pltpu.VMEM_SHARED((DR, DW), jnp.float32),
        pltpu.SemaphoreType.DMA,
        pltpu.SemaphoreType.DMA,
        pltpu.SemaphoreType.DMA,
        pltpu.SemaphoreType.DMA,
    ],
)
def _edge_pass2(src_hbm, dst_hbm, ex_hbm, den_hbm, h_hbm,
                parts_hbm, den_vm, src_vm, dst_vm, ex_vm,
                coef_vm, rows0_vm, rows1_vm,
                dsub0_vm, dsub1_vm, acc_sh, den_sh, gsem0, gsem1,
                ssem0, ssem1):
    cid = lax.axis_index("c")
    sid = lax.axis_index("s")
    wid = sid * NC + cid
    base = wid * EPW

    @pl.when(sid < DR // 8)
    def _():
        pltpu.sync_copy(den_hbm.at[0, pl.ds(sid * 8, 8), :],
                        den_vm.at[pl.ds(0, 8), :])
        pltpu.sync_copy(den_hbm.at[1, pl.ds(sid * 8, 8), :],
                        den_vm.at[pl.ds(8, 8), :])

        def dadd(i, carry):
            r = i // (DW // 16)
            k = i % (DW // 16)
            sl = pl.ds(k * 16, 16)
            den_vm[r, sl] = den_vm[r, sl] + den_vm[r + 8, sl]
            return carry
        lax.fori_loop(0, 8 * (DW // 16), dadd, 0)
        pltpu.sync_copy(den_vm.at[pl.ds(0, 8), :],
                        den_sh.at[pl.ds(sid * 8, 8), :])

    def zrow(i, carry):
        r = i // (D // 16)
        d = i % (D // 16)
        rows0_vm[r, pl.ds(d * 16, 16)] = jnp.zeros((16,), jnp.float32)
        return carry
    lax.fori_loop(0, ZR * (D // 16), zrow, 0)
    assert ZR <= RB

    def zg(k, carry):
        g = sid + k * NS

        @pl.when(g < NZG)
        def _():
            pltpu.sync_copy(rows0_vm.at[pl.ds(0, ZR), :],
                            acc_sh.at[pl.ds(g * ZR, ZR), :])
        return carry
    lax.fori_loop(0, (NZG + NS - 1) // NS, zg, 0)
    plsc.subcore_barrier()
    pltpu.sync_copy(den_sh, den_vm)

    gbufs = (rows0_vm, rows1_vm)
    dsubs = (dsub0_vm, dsub1_vm)
    gsems = (gsem0, gsem1)
    ssems = (ssem0, ssem1)

    def _scale_and_scatter(sb):
        buf = gbufs[sb % 2]
        dsub = dsubs[sb % 2]
        b0 = sb * RB

        @plsc.parallel_loop(0, RB, unroll=4)
        def _scl(e):
            cvec = plsc.load_gather(
                coef_vm, [jnp.full((16,), b0 + e, jnp.int32)])
            for d in range(D // 16):
                sl = pl.ds(d * 16, 16)
                buf[e, sl] = buf[e, sl] * cvec
        for k in range(RB // 16):
            sl = pl.ds(k * 16, 16)
            dsub[sl] = dst_vm[pl.ds(b0 + k * 16, 16)]
        return pltpu.async_copy(buf, acc_sh.at[dsub], ssems[sb % 2],
                                add=True)

    def chunk_body(c, carry):
        off = base + c * CH2
        pltpu.sync_copy(src_hbm.at[pl.ds(off, CH2)], src_vm)
        pltpu.sync_copy(dst_hbm.at[pl.ds(off, CH2)], dst_vm)
        pltpu.sync_copy(ex_hbm.at[pl.ds(off, CH2)], ex_vm)

        @plsc.parallel_loop(0, CH2 // 16, unroll=4)
        def _cgrp(i):
            sl = pl.ds(i * 16, 16)
            di = dst_vm[sl]
            dn = plsc.load_gather(den_vm, [di >> 7, di & 127])
            coef_vm[sl] = ex_vm[sl] / (dn + 1e-16)

        gd = [None] * NSB
        sd = [None] * NSB
        gd[0] = pltpu.async_copy(
            h_hbm.at[src_vm.at[pl.ds(0, RB)]], gbufs[0], gsems[0])
        for sb in range(NSB):
            if sb + 1 < NSB:
                if sb >= 1:
                    sd[sb - 1].wait()
                gd[sb + 1] = pltpu.async_copy(
                    h_hbm.at[src_vm.at[pl.ds((sb + 1) * RB, RB)]],
                    gbufs[(sb + 1) % 2], gsems[(sb + 1) % 2])
            gd[sb].wait()
            sd[sb] = _scale_and_scatter(sb)
        sd[NSB - 2].wait()
        sd[NSB - 1].wait()
        return carry
    lax.fori_loop(0, EPW // CH2, chunk_body, 0)

    plsc.subcore_barrier()

    def og(k, carry):
        g = sid + k * NS

        @pl.when(g < NZG)
        def _():
            pltpu.sync_copy(acc_sh.at[pl.ds(g * ZR, ZR), :],
                            parts_hbm.at[cid, pl.ds(g * ZR, ZR), :])
        return carry
    lax.fori_loop(0, (NZG + NS - 1) // NS, og, 0)


def kernel(x, edge_index, edge_attr, batch, W, a_src, a_dst, W_e, a_e, b,
           W_lin, b_lin):
    src = edge_index[0]
    dst = edge_index[1]

    AE = _edge_alpha_pre(edge_attr, W_e, a_e)

    Wlin_pad = jnp.zeros((D, 8), jnp.float32).at[:, 0].set(W_lin[:, 0])
    blin_pad = jnp.zeros((1, 8), jnp.float32).at[0, 0].set(b_lin[0])
    bt3 = batch.reshape(N // NB, 1, NB)

    parts = None
    for l in range(L):
        A = jnp.zeros((D, 8), jnp.float32)
        A = A.at[:, 0].set(a_src[l]).at[:, 1].set(a_dst[l])
        if l == 0:
            H, S = _score0(x, W[0], A)
        else:
            H, S = _score(parts[0], parts[1], b[l - 1][None, :], W[l], A)
        ex, denp = _edge_pass1(src, dst, AE[l], S.reshape(N * 8))
        parts = _edge_pass2(src, dst, ex, denp, H)

    out = _final(parts[0], parts[1], b[L - 1][None, :], bt3,
                 Wlin_pad, blin_pad)
    return out[:, 0:1]
